# Initial kernel scaffold; baseline (speedup 1.0000x reference)
import jax
import jax.numpy as jnp
from jax.experimental import pallas as pl


def kernel(x, edge_index, edge_w0, edge_w1, W):
    raise NotImplementedError("write your pallas kernel here")


# speedup vs baseline: 3.9884x; 3.9884x over previous
import functools

import jax
import jax.numpy as jnp
from jax import lax
from jax.experimental import pallas as pl
from jax.experimental.pallas import tpu as pltpu
from jax.experimental.pallas import tpu_sc as plsc

N = 10000
E = 160000
D = 256
H = D // 2

NC = 2
NS = 16
LANES = 16

CHUNK = 80
EDGES_PER_TILE = E // NS
NCHUNK = EDGES_PER_TILE // CHUNK
RCHUNK = 80
NROWCHUNK = N // RCHUNK
ROWCHUNK_ITERS = (NROWCHUNK + NS - 1) // NS

ROW_BLK = 1000


def _tc_body(x_ref, w_ref, out_ref):
    pre = jnp.dot(x_ref[...], w_ref[...], preferred_element_type=jnp.float32)
    mean_v = pre[:, :H]
    var_v = jnp.maximum(pre[:, H:], 0.0)
    mean_v = jnp.where(mean_v > 0, mean_v, jnp.exp(mean_v) - 1.0)
    nw = jnp.exp(-var_v)
    out_ref[0] = mean_v * nw
    out_ref[1] = var_v * (nw * nw)


def _features(x, W):
    return pl.pallas_call(
        _tc_body,
        grid=(N // ROW_BLK,),
        in_specs=[
            pl.BlockSpec((ROW_BLK, D), lambda i: (i, 0)),
            pl.BlockSpec((D, D), lambda i: (0, 0)),
        ],
        out_specs=pl.BlockSpec((2, ROW_BLK, H), lambda i: (0, i, 0)),
        out_shape=jax.ShapeDtypeStruct((2, N, H), jnp.float32),
    )(x, W)


def _sc_kernel(feat, src, dst, w2, out, idx_s, idx_d, wbuf, rows, accum, sem):
    c = lax.axis_index("c")
    s = lax.axis_index("s")

    def zero_row(r, _):
        for j in range(H // LANES):
            rows[r, pl.ds(j * LANES, LANES)] = jnp.zeros((LANES,), jnp.float32)
        return 0

    lax.fori_loop(0, RCHUNK, zero_row, 0)

    def zero_copy(k, _):
        cid = s + NS * k

        @pl.when(cid < NROWCHUNK)
        def _():
            r = pl.multiple_of(cid * RCHUNK, 8)
            pltpu.sync_copy(rows, accum.at[pl.ds(r, RCHUNK)])

        return 0

    lax.fori_loop(0, ROWCHUNK_ITERS, zero_copy, 0)
    plsc.subcore_barrier()

    ebase = s * EDGES_PER_TILE

    def chunk_body(i, _):
        e0 = pl.multiple_of(ebase + i * CHUNK, 8)
        pltpu.sync_copy(src.at[pl.ds(e0, CHUNK)], idx_s)
        pltpu.sync_copy(dst.at[pl.ds(e0, CHUNK)], idx_d)
        pltpu.sync_copy(w2.at[pl.ds(c * E + e0, CHUNK)], wbuf)
        off = jnp.full((LANES,), c * N, jnp.int32)
        for k in range(CHUNK // LANES):
            idx_s[pl.ds(k * LANES, LANES)] = idx_s[pl.ds(k * LANES, LANES)] + off
        pltpu.async_copy(feat.at[idx_s], rows, sem).wait()

        def scale_group(g, _):
            wv = wbuf[pl.ds(g * LANES, LANES)]
            for l in range(LANES):
                ws = wv[l]
                e = g * LANES + l
                for j in range(H // LANES):
                    sl = pl.ds(j * LANES, LANES)
                    rows[e, sl] = rows[e, sl] * ws
            return 0

        lax.fori_loop(0, CHUNK // LANES, scale_group, 0)
        pltpu.sync_copy(rows, accum.at[idx_d], add=True)
        return 0

    lax.fori_loop(0, NCHUNK, chunk_body, 0)
    plsc.subcore_barrier()

    def out_copy(k, _):
        cid = s + NS * k

        @pl.when(cid < NROWCHUNK)
        def _():
            r = pl.multiple_of(cid * RCHUNK, 8)
            pltpu.sync_copy(accum.at[pl.ds(r, RCHUNK)], rows)
            pltpu.sync_copy(rows, out.at[pl.ds(r, RCHUNK), pl.ds(c * H, H)])

        return 0

    lax.fori_loop(0, ROWCHUNK_ITERS, out_copy, 0)


@functools.partial(
    pl.kernel,
    out_type=jax.ShapeDtypeStruct((N, D), jnp.float32),
    mesh=plsc.VectorSubcoreMesh(core_axis_name="c", subcore_axis_name="s"),
    scratch_types=[
        pltpu.VMEM((CHUNK,), jnp.int32),
        pltpu.VMEM((CHUNK,), jnp.int32),
        pltpu.VMEM((CHUNK,), jnp.float32),
        pltpu.VMEM((CHUNK, H), jnp.float32),
        pltpu.VMEM_SHARED((N, H), jnp.float32),
        pltpu.SemaphoreType.DMA,
    ],
)
def _sc_scatter(feat, src, dst, w2, out, *scratch):
    _sc_kernel(feat, src, dst, w2, out, *scratch)


def kernel(x, edge_index, edge_w0, edge_w1, W):
    feat = _features(x, W).reshape(2 * N, H)
    src = edge_index[0]
    dst = edge_index[1]
    w2 = jnp.concatenate([edge_w0, edge_w1])
    return _sc_scatter(feat, src, dst, w2)


# speedup vs baseline: 5.2333x; 1.3121x over previous
import functools

import jax
import jax.numpy as jnp
from jax import lax
from jax.experimental import pallas as pl
from jax.experimental.pallas import tpu as pltpu
from jax.experimental.pallas import tpu_sc as plsc

N = 10000
E = 160000
D = 256
H = D // 2

NC = 2
NS = 16
LANES = 16

CHUNK = 128
NCHUNK_TOTAL = 1280
EP = NCHUNK_TOTAL * CHUNK
CPT = NCHUNK_TOTAL // NS
NPASS = 2
CPP = CPT // NPASS
RCHUNK = 80
NROWCHUNK = N // RCHUNK
ROWCHUNK_ITERS = (NROWCHUNK + NS - 1) // NS

ROW_BLK = 1000


def _tc_body(x_ref, w_ref, out_ref):
    pre = jnp.dot(x_ref[...], w_ref[...], preferred_element_type=jnp.float32)
    mean_v = pre[:, :H]
    var_v = jnp.maximum(pre[:, H:], 0.0)
    mean_v = jnp.where(mean_v > 0, mean_v, jnp.exp(mean_v) - 1.0)
    nw = jnp.exp(-var_v)
    out_ref[0] = mean_v * nw
    out_ref[1] = var_v * (nw * nw)


def _features(x, W):
    return pl.pallas_call(
        _tc_body,
        grid=(N // ROW_BLK,),
        in_specs=[
            pl.BlockSpec((ROW_BLK, D), lambda i: (i, 0)),
            pl.BlockSpec((D, D), lambda i: (0, 0)),
        ],
        out_specs=pl.BlockSpec((2, ROW_BLK, H), lambda i: (0, i, 0)),
        out_shape=jax.ShapeDtypeStruct((2, N, H), jnp.float32),
    )(x, W)


def _sc_kernel(feat, srcs, dst2, w3, out, idx_s, idx_d, wts, rows, accum, gsem):
    c = lax.axis_index("c")
    s = lax.axis_index("s")

    def zero_row(r, _):
        for j in range(H // LANES):
            rows[0, r, pl.ds(j * LANES, LANES)] = jnp.zeros((LANES,), jnp.float32)
        return 0

    lax.fori_loop(0, RCHUNK, zero_row, 0)

    def zero_copy(k, _):
        cid = s + NS * k

        @pl.when(cid < NROWCHUNK)
        def _():
            r = pl.multiple_of(cid * RCHUNK, 8)
            pltpu.sync_copy(rows.at[0, pl.ds(0, RCHUNK)], accum.at[pl.ds(r, RCHUNK)])

        return 0

    lax.fori_loop(0, ROWCHUNK_ITERS, zero_copy, 0)

    plsc.subcore_barrier()

    def issue_gather(k, b):
        pltpu.async_copy(feat.at[idx_s.at[k]], rows.at[b], gsem.at[b])

    def pass_body(p, _):
        ck0 = pl.multiple_of(s * CPT + p * CPP, 8)
        pltpu.sync_copy(srcs.at[c, pl.ds(ck0, CPP)], idx_s)
        pltpu.sync_copy(dst2.at[pl.ds(ck0, CPP)], idx_d)
        pltpu.sync_copy(w3.at[c, pl.ds(ck0, CPP)], wts)
        issue_gather(0, 0)

        def chunk_body(g, _):
            for b in range(2):
                k = 2 * g + b

                @pl.when(k + 1 < CPP)
                def _():
                    issue_gather(k + 1, (b + 1) % 2)

                pltpu.make_async_copy(
                    feat.at[idx_s.at[k]], rows.at[b], gsem.at[b]).wait()

                def scale_group(g2, _):
                    wv = wts[k, pl.ds(g2 * LANES, LANES)]
                    for l in range(LANES):
                        ws = wv[l]
                        e = g2 * LANES + l
                        for j in range(H // LANES):
                            sl = pl.ds(j * LANES, LANES)
                            rows[b, e, sl] = rows[b, e, sl] * ws
                    return 0

                lax.fori_loop(0, CHUNK // LANES, scale_group, 0)
                pltpu.sync_copy(rows.at[b], accum.at[idx_d.at[k]], add=True)
            return 0

        lax.fori_loop(0, CPP // 2, chunk_body, 0)
        return 0

    lax.fori_loop(0, NPASS, pass_body, 0)
    plsc.subcore_barrier()

    def out_copy(k, _):
        cid = s + NS * k

        @pl.when(cid < NROWCHUNK)
        def _():
            r = pl.multiple_of(cid * RCHUNK, 8)
            pltpu.sync_copy(accum.at[pl.ds(r, RCHUNK)], rows.at[0, pl.ds(0, RCHUNK)])
            pltpu.sync_copy(rows.at[0, pl.ds(0, RCHUNK)],
                            out.at[pl.ds(r, RCHUNK), pl.ds(c * H, H)])

        return 0

    lax.fori_loop(0, ROWCHUNK_ITERS, out_copy, 0)


@functools.partial(
    pl.kernel,
    out_type=jax.ShapeDtypeStruct((N, D), jnp.float32),
    mesh=plsc.VectorSubcoreMesh(core_axis_name="c", subcore_axis_name="s"),
    scratch_types=[
        pltpu.VMEM((CPP, CHUNK), jnp.int32),
        pltpu.VMEM((CPP, CHUNK), jnp.int32),
        pltpu.VMEM((CPP, CHUNK), jnp.float32),
        pltpu.VMEM((2, CHUNK, H), jnp.float32),
        pltpu.VMEM_SHARED((N, H), jnp.float32),
        pltpu.SemaphoreType.DMA((2,)),
    ],
)
def _sc_scatter(feat, srcs, dst2, w3, out, *scratch):
    _sc_kernel(feat, srcs, dst2, w3, out, *scratch)


def kernel(x, edge_index, edge_w0, edge_w1, W):
    feat = _features(x, W).reshape(2 * N, H)
    pad = EP - E
    src = jnp.pad(edge_index[0], (0, pad))
    dst = jnp.pad(edge_index[1], (0, pad))
    srcs = jnp.stack([src, src + N]).reshape(2, NCHUNK_TOTAL, CHUNK)
    dst2 = dst.reshape(NCHUNK_TOTAL, CHUNK)
    w3 = jnp.stack([jnp.pad(edge_w0, (0, pad)),
                    jnp.pad(edge_w1, (0, pad))]).reshape(2, NCHUNK_TOTAL, CHUNK)
    return _sc_scatter(feat, srcs, dst2, w3)
